# baseline (speedup 1.0000x reference)
import functools
import jax
import jax.numpy as jnp
from jax import lax
from jax.experimental import pallas as pl
from jax.experimental.pallas import tpu as pltpu
from jax.experimental.pallas import tpu_sc as plsc

N = 10000
F_IN = 128
HID = 128
C = 40
CP = 48
ROWS = 1000

NW = 32
CH = 128
NCH = 80
EPW = CH * NCH
EP = NW * EPW
NCHP = 40
NPAD = 10240
RPS = NPAD // 16

_mesh = plsc.VectorSubcoreMesh(core_axis_name="c", subcore_axis_name="s")


def _sc_deg(dst_pad, ones_rows, zeros_d):
    @functools.partial(
        pl.kernel,
        out_type=jax.ShapeDtypeStruct((2, NPAD, 16), jnp.float32),
        mesh=_mesh,
        compiler_params=pltpu.CompilerParams(use_tc_tiling_on_sc=False),
        scratch_types=[
            pltpu.VMEM((CH,), jnp.int32),
            pltpu.VMEM((CH, 16), jnp.float32),
            pltpu.VMEM_SHARED((NPAD, 16), jnp.float32),
            pltpu.SemaphoreType.DMA,
        ],
    )
    def k(dst_hbm, ones_hbm, zeros_hbm, out_hbm, dvv, ones_v, acc_sh, sem):
        cid = lax.axis_index("c")
        sid = lax.axis_index("s")
        w = sid * 2 + cid
        pltpu.sync_copy(zeros_hbm, acc_sh.at[pl.ds(sid * RPS, RPS)])
        pltpu.sync_copy(ones_hbm, ones_v)
        plsc.subcore_barrier()

        @pl.loop(0, NCH)
        def _(c):
            base = w * EPW + c * CH
            pltpu.sync_copy(dst_hbm.at[pl.ds(base, CH)], dvv)
            pltpu.sync_copy(ones_v, acc_sh.at[dvv], add=True)

        plsc.subcore_barrier()
        pltpu.sync_copy(
            acc_sh.at[pl.ds(sid * RPS, RPS)],
            out_hbm.at[cid, pl.ds(sid * RPS, RPS)],
        )

    return k(dst_pad, ones_rows, zeros_d)


def _make_sc_agg(D):
    @functools.partial(
        pl.kernel,
        out_type=jax.ShapeDtypeStruct((2, NPAD, D), jnp.float32),
        mesh=_mesh,
        compiler_params=pltpu.CompilerParams(use_tc_tiling_on_sc=False),
        scratch_types=[
            pltpu.VMEM((NCHP * CH,), jnp.int32),
            pltpu.VMEM((NCHP * CH,), jnp.int32),
            pltpu.VMEM((CH,), jnp.int32),
            pltpu.VMEM((CH,), jnp.int32),
            pltpu.VMEM((CH,), jnp.int32),
            pltpu.VMEM((CH,), jnp.int32),
            pltpu.VMEM((CH, D), jnp.float32),
            pltpu.VMEM((CH, D), jnp.float32),
            pltpu.VMEM_SHARED((NPAD, D), jnp.float32),
            pltpu.SemaphoreType.DMA,
            pltpu.SemaphoreType.DMA,
        ],
    )
    def k(tab_hbm, src_hbm, dst_hbm, zeros_hbm, out_hbm,
          src_v, dst_v, sva, svb, dva, dvb,
          rows_a, rows_b, acc_sh, sem_a, sem_b):
        cid = lax.axis_index("c")
        sid = lax.axis_index("s")
        w = sid * 2 + cid
        pltpu.sync_copy(zeros_hbm, acc_sh.at[pl.ds(sid * RPS, RPS)])
        plsc.subcore_barrier()

        def stage(idx1d, c, buf):
            for j in range(CH // 16):
                buf[pl.ds(j * 16, 16)] = idx1d[pl.ds(c * CH + j * 16, 16)]

        def gather(buf, rows, sem):
            return pltpu.make_async_copy(tab_hbm.at[buf], rows, sem)

        for p in range(NCH // NCHP):
            base = w * EPW + p * NCHP * CH
            pltpu.sync_copy(src_hbm.at[pl.ds(base, NCHP * CH)], src_v)
            pltpu.sync_copy(dst_hbm.at[pl.ds(base, NCHP * CH)], dst_v)
            stage(src_v, 0, sva)
            gather(sva, rows_a, sem_a).start()
            stage(src_v, 1, svb)
            gather(svb, rows_b, sem_b).start()

            @pl.loop(0, NCHP, step=2)
            def _(c):
                gather(sva, rows_a, sem_a).wait()
                stage(dst_v, c, dva)
                pltpu.sync_copy(rows_a, acc_sh.at[dva], add=True)

                @pl.when(c + 2 < NCHP)
                def _():
                    stage(src_v, c + 2, sva)
                    gather(sva, rows_a, sem_a).start()

                gather(svb, rows_b, sem_b).wait()
                stage(dst_v, c + 1, dvb)
                pltpu.sync_copy(rows_b, acc_sh.at[dvb], add=True)

                @pl.when(c + 3 < NCHP)
                def _():
                    stage(src_v, c + 3, svb)
                    gather(svb, rows_b, sem_b).start()

        plsc.subcore_barrier()
        pltpu.sync_copy(
            acc_sh.at[pl.ds(sid * RPS, RPS)],
            out_hbm.at[cid, pl.ds(sid * RPS, RPS)],
        )

    return k


_sc_agg_h = _make_sc_agg(HID)
_sc_agg_c = _make_sc_agg(CP)


def _mm1_body(x_ref, degp_ref, w_ref, o_ref, dinv_ref):
    deg = 1.0 + degp_ref[0, :, 0] + degp_ref[1, :, 0]
    dinv = lax.rsqrt(deg)[:, None]
    dinv_ref[...] = dinv
    xs = x_ref[...] * dinv
    o_ref[...] = jnp.dot(xs, w_ref[...], preferred_element_type=jnp.float32)


def _tc_mm1(x, degp, W1):
    return pl.pallas_call(
        _mm1_body,
        grid=(N // ROWS,),
        in_specs=[
            pl.BlockSpec((ROWS, F_IN), lambda i: (i, 0)),
            pl.BlockSpec((2, ROWS, 16), lambda i: (0, i, 0)),
            pl.BlockSpec((F_IN, HID), lambda i: (0, 0)),
        ],
        out_specs=[
            pl.BlockSpec((ROWS, HID), lambda i: (i, 0)),
            pl.BlockSpec((ROWS, 1), lambda i: (i, 0)),
        ],
        out_shape=[
            jax.ShapeDtypeStruct((N, HID), jnp.float32),
            jax.ShapeDtypeStruct((N, 1), jnp.float32),
        ],
    )(x, degp, W1)


def _mid_body(s1_ref, h1p_ref, dinv_ref, b1_ref, w2_ref, o_ref):
    dinv = dinv_ref[...]
    agg = dinv * (s1_ref[0] + s1_ref[1] + h1p_ref[...]) + b1_ref[...]
    h = jnp.maximum(agg, 0.0) * dinv
    o_ref[...] = jnp.dot(h, w2_ref[...], preferred_element_type=jnp.float32)


def _tc_mid(S1p, h1p, dinv, b1, W2p):
    return pl.pallas_call(
        _mid_body,
        grid=(N // ROWS,),
        in_specs=[
            pl.BlockSpec((2, ROWS, HID), lambda i: (0, i, 0)),
            pl.BlockSpec((ROWS, HID), lambda i: (i, 0)),
            pl.BlockSpec((ROWS, 1), lambda i: (i, 0)),
            pl.BlockSpec((1, HID), lambda i: (0, 0)),
            pl.BlockSpec((HID, CP), lambda i: (0, 0)),
        ],
        out_specs=pl.BlockSpec((ROWS, CP), lambda i: (i, 0)),
        out_shape=jax.ShapeDtypeStruct((N, CP), jnp.float32),
    )(S1p, h1p, dinv, b1, W2p)


def _fin_body(s2_ref, h2p_ref, dinv_ref, b2_ref, o_ref):
    o = dinv_ref[...] * (
        s2_ref[0, :, :C] + s2_ref[1, :, :C] + h2p_ref[:, :C]
    ) + b2_ref[...]
    m = jnp.max(o, axis=1, keepdims=True)
    e = jnp.exp(o - m)
    lse = m + jnp.log(jnp.sum(e, axis=1, keepdims=True))
    o_ref[...] = o - lse


def _tc_fin(S2p, h2p, dinv, b2):
    return pl.pallas_call(
        _fin_body,
        grid=(N // ROWS,),
        in_specs=[
            pl.BlockSpec((2, ROWS, CP), lambda i: (0, i, 0)),
            pl.BlockSpec((ROWS, CP), lambda i: (i, 0)),
            pl.BlockSpec((ROWS, 1), lambda i: (i, 0)),
            pl.BlockSpec((1, C), lambda i: (0, 0)),
        ],
        out_specs=pl.BlockSpec((ROWS, C), lambda i: (i, 0)),
        out_shape=jax.ShapeDtypeStruct((N, C), jnp.float32),
    )(S2p, h2p, dinv, b2)


def kernel(x, edge_index, W1, b1, W2, b2):
    E = edge_index.shape[1]
    npad_e = EP - E
    src = jnp.concatenate(
        [edge_index[0], jnp.zeros((npad_e,), jnp.int32)])
    dst = jnp.concatenate(
        [edge_index[1], jnp.full((npad_e,), N, jnp.int32)])

    ones_rows = jnp.ones((CH, 16), jnp.float32)
    zeros_d = jnp.zeros((RPS, 16), jnp.float32)
    zeros_h = jnp.zeros((RPS, HID), jnp.float32)
    zeros_c = jnp.zeros((RPS, CP), jnp.float32)
    W2p = jnp.pad(W2, ((0, 0), (0, CP - C)))

    degp = _sc_deg(dst, ones_rows, zeros_d)
    h1p, dinv = _tc_mm1(x, degp, W1)
    S1p = _sc_agg_h(h1p, src, dst, zeros_h)
    h2p = _tc_mid(S1p, h1p, dinv, b1[None, :], W2p)
    S2p = _sc_agg_c(h2p, src, dst, zeros_c)
    return _tc_fin(S2p, h2p, dinv, b2[None, :])


# speedup vs baseline: 1.0005x; 1.0005x over previous
import functools
import jax
import jax.numpy as jnp
from jax import lax
from jax.experimental import pallas as pl
from jax.experimental.pallas import tpu as pltpu
from jax.experimental.pallas import tpu_sc as plsc

N = 10000
F_IN = 128
HID = 128
C = 40
CP = 48
ROWS = 1000

NW = 32
CH = 128
NCH = 80
EPW = CH * NCH
EP = NW * EPW
NCHP = 40
NPAD = 10240
RPS = NPAD // 16

_mesh = plsc.VectorSubcoreMesh(core_axis_name="c", subcore_axis_name="s")


def _sc_deg(dst_pad, ones_rows, zeros_d):
    @functools.partial(
        pl.kernel,
        out_type=jax.ShapeDtypeStruct((2, NPAD, 16), jnp.float32),
        mesh=_mesh,
        compiler_params=pltpu.CompilerParams(use_tc_tiling_on_sc=False),
        scratch_types=[
            pltpu.VMEM((CH,), jnp.int32),
            pltpu.VMEM((CH, 16), jnp.float32),
            pltpu.VMEM_SHARED((NPAD, 16), jnp.float32),
            pltpu.SemaphoreType.DMA,
        ],
    )
    def k(dst_hbm, ones_hbm, zeros_hbm, out_hbm, dvv, ones_v, acc_sh, sem):
        cid = lax.axis_index("c")
        sid = lax.axis_index("s")
        w = sid * 2 + cid
        pltpu.sync_copy(zeros_hbm, acc_sh.at[pl.ds(sid * RPS, RPS)])
        pltpu.sync_copy(ones_hbm, ones_v)
        plsc.subcore_barrier()

        @pl.loop(0, NCH)
        def _(c):
            base = w * EPW + c * CH
            pltpu.sync_copy(dst_hbm.at[pl.ds(base, CH)], dvv)
            pltpu.sync_copy(ones_v, acc_sh.at[dvv], add=True)

        plsc.subcore_barrier()
        pltpu.sync_copy(
            acc_sh.at[pl.ds(sid * RPS, RPS)],
            out_hbm.at[cid, pl.ds(sid * RPS, RPS)],
        )

    return k(dst_pad, ones_rows, zeros_d)


def _make_sc_agg(D):
    @functools.partial(
        pl.kernel,
        out_type=jax.ShapeDtypeStruct((2, NPAD, D), jnp.float32),
        mesh=_mesh,
        compiler_params=pltpu.CompilerParams(use_tc_tiling_on_sc=False),
        scratch_types=[
            pltpu.VMEM((NCHP * CH,), jnp.int32),
            pltpu.VMEM((NCHP * CH,), jnp.int32),
            pltpu.VMEM((CH,), jnp.int32),
            pltpu.VMEM((CH,), jnp.int32),
            pltpu.VMEM((CH,), jnp.int32),
            pltpu.VMEM((CH,), jnp.int32),
            pltpu.VMEM((CH, D), jnp.float32),
            pltpu.VMEM((CH, D), jnp.float32),
            pltpu.VMEM_SHARED((NPAD, D), jnp.float32),
            pltpu.SemaphoreType.DMA,
            pltpu.SemaphoreType.DMA,
        ],
    )
    def k(tab_hbm, src_hbm, dst_hbm, zeros_hbm, out_hbm,
          src_v, dst_v, sva, svb, dva, dvb,
          rows_a, rows_b, acc_sh, sem_a, sem_b):
        cid = lax.axis_index("c")
        sid = lax.axis_index("s")
        w = sid * 2 + cid
        pltpu.sync_copy(zeros_hbm, acc_sh.at[pl.ds(sid * RPS, RPS)])
        plsc.subcore_barrier()

        def stage(idx1d, c, buf):
            for j in range(CH // 16):
                buf[pl.ds(j * 16, 16)] = idx1d[pl.ds(c * CH + j * 16, 16)]

        def gather(buf, rows, sem):
            return pltpu.make_async_copy(tab_hbm.at[buf], rows, sem)

        for p in range(NCH // NCHP):
            base = w * EPW + p * NCHP * CH
            pltpu.sync_copy(src_hbm.at[pl.ds(base, NCHP * CH)], src_v)
            pltpu.sync_copy(dst_hbm.at[pl.ds(base, NCHP * CH)], dst_v)
            stage(src_v, 0, sva)
            gather(sva, rows_a, sem_a).start()
            stage(src_v, 1, svb)
            gather(svb, rows_b, sem_b).start()

            @pl.loop(0, NCHP, step=2)
            def _(c):
                gather(sva, rows_a, sem_a).wait()
                stage(dst_v, c, dva)
                pltpu.sync_copy(rows_a, acc_sh.at[dva], add=True)

                @pl.when(c + 2 < NCHP)
                def _():
                    stage(src_v, c + 2, sva)
                    gather(sva, rows_a, sem_a).start()

                gather(svb, rows_b, sem_b).wait()
                stage(dst_v, c + 1, dvb)
                pltpu.sync_copy(rows_b, acc_sh.at[dvb], add=True)

                @pl.when(c + 3 < NCHP)
                def _():
                    stage(src_v, c + 3, svb)
                    gather(svb, rows_b, sem_b).start()

        plsc.subcore_barrier()
        pltpu.sync_copy(
            acc_sh.at[pl.ds(sid * RPS, RPS)],
            out_hbm.at[cid, pl.ds(sid * RPS, RPS)],
        )

    return k


_sc_agg_h = _make_sc_agg(HID)
_sc_agg_c = _make_sc_agg(CP)


def _mm1_body(x_ref, degp_ref, w_ref, o_ref, dinv_ref):
    deg = 1.0 + degp_ref[0, :, 0] + degp_ref[1, :, 0]
    dinv = lax.rsqrt(deg)[:, None]
    dinv_ref[...] = dinv
    xs = x_ref[...] * dinv
    o_ref[...] = jnp.dot(xs, w_ref[...], preferred_element_type=jnp.float32)


def _tc_mm1(x, degp, W1):
    return pl.pallas_call(
        _mm1_body,
        grid=(N // ROWS,),
        in_specs=[
            pl.BlockSpec((ROWS, F_IN), lambda i: (i, 0)),
            pl.BlockSpec((2, ROWS, 16), lambda i: (0, i, 0)),
            pl.BlockSpec((F_IN, HID), lambda i: (0, 0)),
        ],
        out_specs=[
            pl.BlockSpec((ROWS, HID), lambda i: (i, 0)),
            pl.BlockSpec((ROWS, 1), lambda i: (i, 0)),
        ],
        out_shape=[
            jax.ShapeDtypeStruct((N, HID), jnp.float32),
            jax.ShapeDtypeStruct((N, 1), jnp.float32),
        ],
    )(x, degp, W1)


def _mid_body(s1_ref, h1p_ref, dinv_ref, b1_ref, w2_ref, o_ref):
    dinv = dinv_ref[...]
    agg = dinv * (s1_ref[0] + s1_ref[1] + h1p_ref[...]) + b1_ref[...]
    h = jnp.maximum(agg, 0.0) * dinv
    o_ref[...] = jnp.dot(h, w2_ref[...], preferred_element_type=jnp.float32)


def _tc_mid(S1p, h1p, dinv, b1, W2p):
    return pl.pallas_call(
        _mid_body,
        grid=(N // ROWS,),
        in_specs=[
            pl.BlockSpec((2, ROWS, HID), lambda i: (0, i, 0)),
            pl.BlockSpec((ROWS, HID), lambda i: (i, 0)),
            pl.BlockSpec((ROWS, 1), lambda i: (i, 0)),
            pl.BlockSpec((1, HID), lambda i: (0, 0)),
            pl.BlockSpec((HID, CP), lambda i: (0, 0)),
        ],
        out_specs=pl.BlockSpec((ROWS, CP), lambda i: (i, 0)),
        out_shape=jax.ShapeDtypeStruct((N, CP), jnp.float32),
    )(S1p, h1p, dinv, b1, W2p)


def _fin_body(s2_ref, h2p_ref, dinv_ref, b2_ref, o_ref):
    o = dinv_ref[...] * (
        s2_ref[0, :, :C] + s2_ref[1, :, :C] + h2p_ref[:, :C]
    ) + b2_ref[...]
    m = jnp.max(o, axis=1, keepdims=True)
    e = jnp.exp(o - m)
    lse = m + jnp.log(jnp.sum(e, axis=1, keepdims=True))
    o_ref[...] = o - lse


def _tc_fin(S2p, h2p, dinv, b2):
    return pl.pallas_call(
        _fin_body,
        grid=(N // ROWS,),
        in_specs=[
            pl.BlockSpec((2, ROWS, CP), lambda i: (0, i, 0)),
            pl.BlockSpec((ROWS, CP), lambda i: (i, 0)),
            pl.BlockSpec((ROWS, 1), lambda i: (i, 0)),
            pl.BlockSpec((1, C), lambda i: (0, 0)),
        ],
        out_specs=pl.BlockSpec((ROWS, C), lambda i: (i, 0)),
        out_shape=jax.ShapeDtypeStruct((N, C), jnp.float32),
    )(S2p, h2p, dinv, b2)


def kernel(x, edge_index, W1, b1, W2, b2):
    E = edge_index.shape[1]
    npad_e = EP - E
    pad_dst = N + jnp.arange(npad_e, dtype=jnp.int32) % (NPAD - N)
    src = jnp.concatenate(
        [edge_index[0], jnp.zeros((npad_e,), jnp.int32)])
    dst = jnp.concatenate([edge_index[1], pad_dst])

    ones_rows = jnp.ones((CH, 16), jnp.float32)
    zeros_d = jnp.zeros((RPS, 16), jnp.float32)
    zeros_h = jnp.zeros((RPS, HID), jnp.float32)
    zeros_c = jnp.zeros((RPS, CP), jnp.float32)
    W2p = jnp.pad(W2, ((0, 0), (0, CP - C)))

    degp = _sc_deg(dst, ones_rows, zeros_d)
    h1p, dinv = _tc_mm1(x, degp, W1)
    S1p = _sc_agg_h(h1p, src, dst, zeros_h)
    h2p = _tc_mid(S1p, h1p, dinv, b1[None, :], W2p)
    S2p = _sc_agg_c(h2p, src, dst, zeros_c)
    return _tc_fin(S2p, h2p, dinv, b2[None, :])


# speedup vs baseline: 1.1532x; 1.1526x over previous
import functools
import jax
import jax.numpy as jnp
from jax import lax
from jax.experimental import pallas as pl
from jax.experimental.pallas import tpu as pltpu
from jax.experimental.pallas import tpu_sc as plsc

N = 10000
F_IN = 128
HID = 128
C = 40
CP = 48
ROWS = 1000

NW = 32
CH = 128
NCH = 80
EPW = CH * NCH
EP = NW * EPW
NCHP = 40
NPAD = 10240
RPS = NPAD // 16

_mesh = plsc.VectorSubcoreMesh(core_axis_name="c", subcore_axis_name="s")


def _sc_deg(dst_pad, ones_rows, zeros_d):
    @functools.partial(
        pl.kernel,
        out_type=jax.ShapeDtypeStruct((2, NPAD, 16), jnp.float32),
        mesh=_mesh,
        compiler_params=pltpu.CompilerParams(use_tc_tiling_on_sc=False),
        scratch_types=[
            pltpu.VMEM((CH,), jnp.int32),
            pltpu.VMEM((CH, 16), jnp.float32),
            pltpu.VMEM_SHARED((NPAD, 16), jnp.float32),
            pltpu.SemaphoreType.DMA,
        ],
    )
    def k(dst_hbm, ones_hbm, zeros_hbm, out_hbm, dvv, ones_v, acc_sh, sem):
        cid = lax.axis_index("c")
        sid = lax.axis_index("s")
        w = sid * 2 + cid
        pltpu.sync_copy(zeros_hbm, acc_sh.at[pl.ds(sid * RPS, RPS)])
        pltpu.sync_copy(ones_hbm, ones_v)
        plsc.subcore_barrier()

        @pl.loop(0, NCH)
        def _(c):
            base = w * EPW + c * CH
            pltpu.sync_copy(dst_hbm.at[pl.ds(base, CH)], dvv)
            pltpu.sync_copy(ones_v, acc_sh.at[dvv], add=True)

        plsc.subcore_barrier()
        pltpu.sync_copy(
            acc_sh.at[pl.ds(sid * RPS, RPS)],
            out_hbm.at[cid, pl.ds(sid * RPS, RPS)],
        )

    return k(dst_pad, ones_rows, zeros_d)


def _make_sc_agg(D, tab_in_spmem=False):
    @functools.partial(
        pl.kernel,
        out_type=jax.ShapeDtypeStruct((2, NPAD, D), jnp.float32),
        mesh=_mesh,
        compiler_params=pltpu.CompilerParams(use_tc_tiling_on_sc=False),
        scratch_types=[
            pltpu.VMEM((NCHP * CH,), jnp.int32),
            pltpu.VMEM((NCHP * CH,), jnp.int32),
            pltpu.VMEM((CH,), jnp.int32),
            pltpu.VMEM((CH,), jnp.int32),
            pltpu.VMEM((CH,), jnp.int32),
            pltpu.VMEM((CH,), jnp.int32),
            pltpu.VMEM((CH, D), jnp.float32),
            pltpu.VMEM((CH, D), jnp.float32),
            pltpu.VMEM_SHARED((NPAD, D), jnp.float32),
        ]
        + ([pltpu.VMEM_SHARED((N, D), jnp.float32)] if tab_in_spmem else [])
        + [
            pltpu.SemaphoreType.DMA,
            pltpu.SemaphoreType.DMA,
        ],
    )
    def k(tab_hbm, src_hbm, dst_hbm, zeros_hbm, out_hbm,
          src_v, dst_v, sva, svb, dva, dvb,
          rows_a, rows_b, acc_sh, *rest):
        if tab_in_spmem:
            tab_sh, sem_a, sem_b = rest
        else:
            sem_a, sem_b = rest
        cid = lax.axis_index("c")
        sid = lax.axis_index("s")
        w = sid * 2 + cid
        pltpu.sync_copy(zeros_hbm, acc_sh.at[pl.ds(sid * RPS, RPS)])
        if tab_in_spmem:
            nrs = N // 16
            pltpu.sync_copy(
                tab_hbm.at[pl.ds(sid * nrs, nrs)],
                tab_sh.at[pl.ds(sid * nrs, nrs)],
            )
            tab = tab_sh
        else:
            tab = tab_hbm
        plsc.subcore_barrier()

        def stage(idx1d, c, buf):
            for j in range(CH // 16):
                buf[pl.ds(j * 16, 16)] = idx1d[pl.ds(c * CH + j * 16, 16)]

        def gather(buf, rows, sem):
            return pltpu.make_async_copy(tab.at[buf], rows, sem)

        for p in range(NCH // NCHP):
            base = w * EPW + p * NCHP * CH
            pltpu.sync_copy(src_hbm.at[pl.ds(base, NCHP * CH)], src_v)
            pltpu.sync_copy(dst_hbm.at[pl.ds(base, NCHP * CH)], dst_v)
            stage(src_v, 0, sva)
            gather(sva, rows_a, sem_a).start()
            stage(src_v, 1, svb)
            gather(svb, rows_b, sem_b).start()

            @pl.loop(0, NCHP, step=2)
            def _(c):
                gather(sva, rows_a, sem_a).wait()
                stage(dst_v, c, dva)
                pltpu.sync_copy(rows_a, acc_sh.at[dva], add=True)

                @pl.when(c + 2 < NCHP)
                def _():
                    stage(src_v, c + 2, sva)
                    gather(sva, rows_a, sem_a).start()

                gather(svb, rows_b, sem_b).wait()
                stage(dst_v, c + 1, dvb)
                pltpu.sync_copy(rows_b, acc_sh.at[dvb], add=True)

                @pl.when(c + 3 < NCHP)
                def _():
                    stage(src_v, c + 3, svb)
                    gather(svb, rows_b, sem_b).start()

        plsc.subcore_barrier()
        pltpu.sync_copy(
            acc_sh.at[pl.ds(sid * RPS, RPS)],
            out_hbm.at[cid, pl.ds(sid * RPS, RPS)],
        )

    return k


_sc_agg_h = _make_sc_agg(HID)
_sc_agg_c = _make_sc_agg(CP, tab_in_spmem=True)


def _mm1_body(x_ref, degp_ref, w_ref, o_ref, dinv_ref):
    deg = 1.0 + degp_ref[0, :, 0] + degp_ref[1, :, 0]
    dinv = lax.rsqrt(deg)[:, None]
    dinv_ref[...] = dinv
    xs = x_ref[...] * dinv
    o_ref[...] = jnp.dot(xs, w_ref[...], preferred_element_type=jnp.float32)


def _tc_mm1(x, degp, W1):
    return pl.pallas_call(
        _mm1_body,
        grid=(N // ROWS,),
        in_specs=[
            pl.BlockSpec((ROWS, F_IN), lambda i: (i, 0)),
            pl.BlockSpec((2, ROWS, 16), lambda i: (0, i, 0)),
            pl.BlockSpec((F_IN, HID), lambda i: (0, 0)),
        ],
        out_specs=[
            pl.BlockSpec((ROWS, HID), lambda i: (i, 0)),
            pl.BlockSpec((ROWS, 1), lambda i: (i, 0)),
        ],
        out_shape=[
            jax.ShapeDtypeStruct((N, HID), jnp.float32),
            jax.ShapeDtypeStruct((N, 1), jnp.float32),
        ],
    )(x, degp, W1)


def _mid_body(s1_ref, h1p_ref, dinv_ref, b1_ref, w2_ref, o_ref):
    dinv = dinv_ref[...]
    agg = dinv * (s1_ref[0] + s1_ref[1] + h1p_ref[...]) + b1_ref[...]
    h = jnp.maximum(agg, 0.0) * dinv
    o_ref[...] = jnp.dot(h, w2_ref[...], preferred_element_type=jnp.float32)


def _tc_mid(S1p, h1p, dinv, b1, W2p):
    return pl.pallas_call(
        _mid_body,
        grid=(N // ROWS,),
        in_specs=[
            pl.BlockSpec((2, ROWS, HID), lambda i: (0, i, 0)),
            pl.BlockSpec((ROWS, HID), lambda i: (i, 0)),
            pl.BlockSpec((ROWS, 1), lambda i: (i, 0)),
            pl.BlockSpec((1, HID), lambda i: (0, 0)),
            pl.BlockSpec((HID, CP), lambda i: (0, 0)),
        ],
        out_specs=pl.BlockSpec((ROWS, CP), lambda i: (i, 0)),
        out_shape=jax.ShapeDtypeStruct((N, CP), jnp.float32),
    )(S1p, h1p, dinv, b1, W2p)


def _fin_body(s2_ref, h2p_ref, dinv_ref, b2_ref, o_ref):
    o = dinv_ref[...] * (
        s2_ref[0, :, :C] + s2_ref[1, :, :C] + h2p_ref[:, :C]
    ) + b2_ref[...]
    m = jnp.max(o, axis=1, keepdims=True)
    e = jnp.exp(o - m)
    lse = m + jnp.log(jnp.sum(e, axis=1, keepdims=True))
    o_ref[...] = o - lse


def _tc_fin(S2p, h2p, dinv, b2):
    return pl.pallas_call(
        _fin_body,
        grid=(N // ROWS,),
        in_specs=[
            pl.BlockSpec((2, ROWS, CP), lambda i: (0, i, 0)),
            pl.BlockSpec((ROWS, CP), lambda i: (i, 0)),
            pl.BlockSpec((ROWS, 1), lambda i: (i, 0)),
            pl.BlockSpec((1, C), lambda i: (0, 0)),
        ],
        out_specs=pl.BlockSpec((ROWS, C), lambda i: (i, 0)),
        out_shape=jax.ShapeDtypeStruct((N, C), jnp.float32),
    )(S2p, h2p, dinv, b2)


def kernel(x, edge_index, W1, b1, W2, b2):
    E = edge_index.shape[1]
    npad_e = EP - E
    pad_dst = N + jnp.arange(npad_e, dtype=jnp.int32) % (NPAD - N)
    src = jnp.concatenate(
        [edge_index[0], jnp.zeros((npad_e,), jnp.int32)])
    dst = jnp.concatenate([edge_index[1], pad_dst])

    ones_rows = jnp.ones((CH, 16), jnp.float32)
    zeros_d = jnp.zeros((RPS, 16), jnp.float32)
    zeros_h = jnp.zeros((RPS, HID), jnp.float32)
    zeros_c = jnp.zeros((RPS, CP), jnp.float32)
    W2p = jnp.pad(W2, ((0, 0), (0, CP - C)))

    degp = _sc_deg(dst, ones_rows, zeros_d)
    h1p, dinv = _tc_mm1(x, degp, W1)
    S1p = _sc_agg_h(h1p, src, dst, zeros_h)
    h2p = _tc_mid(S1p, h1p, dinv, b1[None, :], W2p)
    S2p = _sc_agg_c(h2p, src, dst, zeros_c)
    return _tc_fin(S2p, h2p, dinv, b2[None, :])


# speedup vs baseline: 1.8135x; 1.5726x over previous
import functools
import jax
import jax.numpy as jnp
from jax import lax
from jax.experimental import pallas as pl
from jax.experimental.pallas import tpu as pltpu
from jax.experimental.pallas import tpu_sc as plsc

N = 10000
F_IN = 128
HID = 128
C = 40
CP = 48
ROWS = 1000

NW = 32
CH = 128
NCH = 80
EPW = CH * NCH
EP = NW * EPW
NCHP = 40
NPAD = 10240
RPS = NPAD // 16

_mesh = plsc.VectorSubcoreMesh(core_axis_name="c", subcore_axis_name="s")


def _sc_deg(dst_pad, ones_rows, zeros_d):
    @functools.partial(
        pl.kernel,
        out_type=jax.ShapeDtypeStruct((2, NPAD, 16), jnp.float32),
        mesh=_mesh,
        compiler_params=pltpu.CompilerParams(use_tc_tiling_on_sc=False),
        scratch_types=[
            pltpu.VMEM((CH,), jnp.int32),
            pltpu.VMEM((CH, 16), jnp.float32),
            pltpu.VMEM_SHARED((NPAD, 16), jnp.float32),
            pltpu.SemaphoreType.DMA,
        ],
    )
    def k(dst_hbm, ones_hbm, zeros_hbm, out_hbm, dvv, ones_v, acc_sh, sem):
        cid = lax.axis_index("c")
        sid = lax.axis_index("s")
        w = sid * 2 + cid
        pltpu.sync_copy(zeros_hbm, acc_sh.at[pl.ds(sid * RPS, RPS)])
        pltpu.sync_copy(ones_hbm, ones_v)
        plsc.subcore_barrier()

        @pl.loop(0, NCH)
        def _(c):
            base = w * EPW + c * CH
            pltpu.sync_copy(dst_hbm.at[pl.ds(base, CH)], dvv)
            pltpu.sync_copy(ones_v, acc_sh.at[dvv], add=True)

        plsc.subcore_barrier()
        pltpu.sync_copy(
            acc_sh.at[pl.ds(sid * RPS, RPS)],
            out_hbm.at[cid, pl.ds(sid * RPS, RPS)],
        )

    return k(dst_pad, ones_rows, zeros_d)


def _make_sc_agg(D, tab_in_spmem=False):
    @functools.partial(
        pl.kernel,
        out_type=jax.ShapeDtypeStruct((2, NPAD, D), jnp.float32),
        mesh=_mesh,
        compiler_params=pltpu.CompilerParams(use_tc_tiling_on_sc=False),
        scratch_types=[
            pltpu.VMEM((NCHP * CH,), jnp.int32),
            pltpu.VMEM((NCHP * CH,), jnp.int32),
            pltpu.VMEM((CH,), jnp.int32),
            pltpu.VMEM((CH,), jnp.int32),
            pltpu.VMEM((CH,), jnp.int32),
            pltpu.VMEM((CH,), jnp.int32),
            pltpu.VMEM((CH, D), jnp.float32),
            pltpu.VMEM((CH, D), jnp.float32),
            pltpu.VMEM_SHARED((NPAD, D), jnp.float32),
        ]
        + ([pltpu.VMEM_SHARED((N, D), jnp.float32)] if tab_in_spmem else [])
        + [
            pltpu.SemaphoreType.DMA,
            pltpu.SemaphoreType.DMA,
        ],
    )
    def k(tab_hbm, src_hbm, dst_hbm, zeros_hbm, out_hbm,
          src_v, dst_v, sva, svb, dva, dvb,
          rows_a, rows_b, acc_sh, *rest):
        if tab_in_spmem:
            tab_sh, sem_a, sem_b = rest
        else:
            sem_a, sem_b = rest
        cid = lax.axis_index("c")
        sid = lax.axis_index("s")
        w = sid * 2 + cid
        pltpu.sync_copy(zeros_hbm, acc_sh.at[pl.ds(sid * RPS, RPS)])
        if tab_in_spmem:
            nrs = N // 16
            pltpu.sync_copy(
                tab_hbm.at[pl.ds(sid * nrs, nrs)],
                tab_sh.at[pl.ds(sid * nrs, nrs)],
            )
            tab = tab_sh
        else:
            tab = tab_hbm
        plsc.subcore_barrier()

        def stage(idx1d, c, buf):
            for j in range(CH // 16):
                buf[pl.ds(j * 16, 16)] = idx1d[pl.ds(c * CH + j * 16, 16)]

        def gather(buf, rows, sem):
            return pltpu.make_async_copy(tab.at[buf], rows, sem)

        for p in range(NCH // NCHP):
            base = w * EPW + p * NCHP * CH
            pltpu.sync_copy(src_hbm.at[pl.ds(base, NCHP * CH)], src_v)
            pltpu.sync_copy(dst_hbm.at[pl.ds(base, NCHP * CH)], dst_v)
            stage(src_v, 0, sva)
            gather(sva, rows_a, sem_a).start()
            stage(src_v, 1, svb)
            gather(svb, rows_b, sem_b).start()

            @pl.loop(0, NCHP, step=2)
            def _(c):
                gather(sva, rows_a, sem_a).wait()
                stage(dst_v, c, dva)
                pltpu.sync_copy(rows_a, acc_sh.at[dva], add=True)

                @pl.when(c + 2 < NCHP)
                def _():
                    stage(src_v, c + 2, sva)
                    gather(sva, rows_a, sem_a).start()

                gather(svb, rows_b, sem_b).wait()
                stage(dst_v, c + 1, dvb)
                pltpu.sync_copy(rows_b, acc_sh.at[dvb], add=True)

                @pl.when(c + 3 < NCHP)
                def _():
                    stage(src_v, c + 3, svb)
                    gather(svb, rows_b, sem_b).start()

        plsc.subcore_barrier()
        pltpu.sync_copy(
            acc_sh.at[pl.ds(sid * RPS, RPS)],
            out_hbm.at[cid, pl.ds(sid * RPS, RPS)],
        )

    return k


_sc_agg_c = _make_sc_agg(CP, tab_in_spmem=True)

HH = HID // 2
EPC = EP // 16
NCHL = EPC // CH


def _sc_agg_split():
    @functools.partial(
        pl.kernel,
        out_type=jax.ShapeDtypeStruct((2, NPAD, HH), jnp.float32),
        mesh=_mesh,
        compiler_params=pltpu.CompilerParams(use_tc_tiling_on_sc=False),
        scratch_types=[
            pltpu.VMEM((NCHP * CH,), jnp.int32),
            pltpu.VMEM((NCHP * CH,), jnp.int32),
            pltpu.VMEM((CH,), jnp.int32),
            pltpu.VMEM((CH,), jnp.int32),
            pltpu.VMEM((CH,), jnp.int32),
            pltpu.VMEM((CH,), jnp.int32),
            pltpu.VMEM((CH, HH), jnp.float32),
            pltpu.VMEM((CH, HH), jnp.float32),
            pltpu.VMEM_SHARED((NPAD, HH), jnp.float32),
            pltpu.VMEM_SHARED((N, HH), jnp.float32),
            pltpu.SemaphoreType.DMA,
            pltpu.SemaphoreType.DMA,
        ],
    )
    def k(tabs_hbm, src_hbm, dst_hbm, zeros_hbm, out_hbm,
          src_v, dst_v, sva, svb, dva, dvb,
          rows_a, rows_b, acc_sh, tab_sh, sem_a, sem_b):
        cid = lax.axis_index("c")
        sid = lax.axis_index("s")
        pltpu.sync_copy(zeros_hbm, acc_sh.at[pl.ds(sid * RPS, RPS)])
        nrs = N // 16
        pltpu.sync_copy(
            tabs_hbm.at[cid, pl.ds(sid * nrs, nrs)],
            tab_sh.at[pl.ds(sid * nrs, nrs)],
        )
        plsc.subcore_barrier()

        def stage(idx1d, c, buf):
            for j in range(CH // 16):
                buf[pl.ds(j * 16, 16)] = idx1d[pl.ds(c * CH + j * 16, 16)]

        def gather(buf, rows, sem):
            return pltpu.make_async_copy(tab_sh.at[buf], rows, sem)

        for p in range(NCHL // NCHP):
            base = sid * EPC + p * NCHP * CH
            pltpu.sync_copy(src_hbm.at[pl.ds(base, NCHP * CH)], src_v)
            pltpu.sync_copy(dst_hbm.at[pl.ds(base, NCHP * CH)], dst_v)
            stage(src_v, 0, sva)
            gather(sva, rows_a, sem_a).start()
            stage(src_v, 1, svb)
            gather(svb, rows_b, sem_b).start()

            @pl.loop(0, NCHP, step=2)
            def _(c):
                gather(sva, rows_a, sem_a).wait()
                stage(dst_v, c, dva)
                pltpu.sync_copy(rows_a, acc_sh.at[dva], add=True)

                @pl.when(c + 2 < NCHP)
                def _():
                    stage(src_v, c + 2, sva)
                    gather(sva, rows_a, sem_a).start()

                gather(svb, rows_b, sem_b).wait()
                stage(dst_v, c + 1, dvb)
                pltpu.sync_copy(rows_b, acc_sh.at[dvb], add=True)

                @pl.when(c + 3 < NCHP)
                def _():
                    stage(src_v, c + 3, svb)
                    gather(svb, rows_b, sem_b).start()

        plsc.subcore_barrier()
        pltpu.sync_copy(
            acc_sh.at[pl.ds(sid * RPS, RPS)],
            out_hbm.at[cid, pl.ds(sid * RPS, RPS)],
        )

    return k


_sc_agg_h_split = _sc_agg_split()


def _mm1_body(x_ref, degp_ref, w_ref, o_ref, dinv_ref):
    deg = 1.0 + degp_ref[0, :, 0] + degp_ref[1, :, 0]
    dinv = lax.rsqrt(deg)[:, None]
    dinv_ref[...] = dinv
    xs = x_ref[...] * dinv
    o_ref[...] = jnp.dot(xs, w_ref[...], preferred_element_type=jnp.float32)


def _tc_mm1(x, degp, W1):
    return pl.pallas_call(
        _mm1_body,
        grid=(N // ROWS,),
        in_specs=[
            pl.BlockSpec((ROWS, F_IN), lambda i: (i, 0)),
            pl.BlockSpec((2, ROWS, 16), lambda i: (0, i, 0)),
            pl.BlockSpec((F_IN, HID), lambda i: (0, 0)),
        ],
        out_specs=[
            pl.BlockSpec((ROWS, HID), lambda i: (i, 0)),
            pl.BlockSpec((ROWS, 1), lambda i: (i, 0)),
        ],
        out_shape=[
            jax.ShapeDtypeStruct((N, HID), jnp.float32),
            jax.ShapeDtypeStruct((N, 1), jnp.float32),
        ],
    )(x, degp, W1)


def _mid_body(s1_ref, h1p_ref, dinv_ref, b1_ref, w2_ref, o_ref):
    dinv = dinv_ref[...]
    s1 = jnp.concatenate([s1_ref[0], s1_ref[1]], axis=1)
    agg = dinv * (s1 + h1p_ref[...]) + b1_ref[...]
    h = jnp.maximum(agg, 0.0) * dinv
    o_ref[...] = jnp.dot(h, w2_ref[...], preferred_element_type=jnp.float32)


def _tc_mid(S1p, h1p, dinv, b1, W2p):
    return pl.pallas_call(
        _mid_body,
        grid=(N // ROWS,),
        in_specs=[
            pl.BlockSpec((2, ROWS, HH), lambda i: (0, i, 0)),
            pl.BlockSpec((ROWS, HID), lambda i: (i, 0)),
            pl.BlockSpec((ROWS, 1), lambda i: (i, 0)),
            pl.BlockSpec((1, HID), lambda i: (0, 0)),
            pl.BlockSpec((HID, CP), lambda i: (0, 0)),
        ],
        out_specs=pl.BlockSpec((ROWS, CP), lambda i: (i, 0)),
        out_shape=jax.ShapeDtypeStruct((N, CP), jnp.float32),
    )(S1p, h1p, dinv, b1, W2p)


def _fin_body(s2_ref, h2p_ref, dinv_ref, b2_ref, o_ref):
    o = dinv_ref[...] * (
        s2_ref[0, :, :C] + s2_ref[1, :, :C] + h2p_ref[:, :C]
    ) + b2_ref[...]
    m = jnp.max(o, axis=1, keepdims=True)
    e = jnp.exp(o - m)
    lse = m + jnp.log(jnp.sum(e, axis=1, keepdims=True))
    o_ref[...] = o - lse


def _tc_fin(S2p, h2p, dinv, b2):
    return pl.pallas_call(
        _fin_body,
        grid=(N // ROWS,),
        in_specs=[
            pl.BlockSpec((2, ROWS, CP), lambda i: (0, i, 0)),
            pl.BlockSpec((ROWS, CP), lambda i: (i, 0)),
            pl.BlockSpec((ROWS, 1), lambda i: (i, 0)),
            pl.BlockSpec((1, C), lambda i: (0, 0)),
        ],
        out_specs=pl.BlockSpec((ROWS, C), lambda i: (i, 0)),
        out_shape=jax.ShapeDtypeStruct((N, C), jnp.float32),
    )(S2p, h2p, dinv, b2)


def kernel(x, edge_index, W1, b1, W2, b2):
    E = edge_index.shape[1]
    npad_e = EP - E
    pad_dst = N + jnp.arange(npad_e, dtype=jnp.int32) % (NPAD - N)
    src = jnp.concatenate(
        [edge_index[0], jnp.zeros((npad_e,), jnp.int32)])
    dst = jnp.concatenate([edge_index[1], pad_dst])

    ones_rows = jnp.ones((CH, 16), jnp.float32)
    zeros_d = jnp.zeros((RPS, 16), jnp.float32)
    zeros_h = jnp.zeros((RPS, HH), jnp.float32)
    zeros_c = jnp.zeros((RPS, CP), jnp.float32)
    W2p = jnp.pad(W2, ((0, 0), (0, CP - C)))

    degp = _sc_deg(dst, ones_rows, zeros_d)
    h1p, dinv = _tc_mm1(x, degp, W1)
    h1s = jnp.stack([h1p[:, :HH], h1p[:, HH:]])
    S1p = _sc_agg_h_split(h1s, src, dst, zeros_h)
    h2p = _tc_mid(S1p, h1p, dinv, b1[None, :], W2p)
    S2p = _sc_agg_c(h2p, src, dst, zeros_c)
    return _tc_fin(S2p, h2p, dinv, b2[None, :])
